# baseline (speedup 1.0000x reference)
import functools

import jax
import jax.numpy as jnp
from jax import lax
from jax.experimental import pallas as pl
from jax.experimental.pallas import tpu as pltpu
from jax.experimental.pallas import tpu_sc as plsc

_NUM_LABELS = 1000
_CLAMP = 4.6051
_B = 16384
_D = 128
_LPAD = 1024
_BLK = 512
_NSTEPS = _B // _BLK
_NC = 2
_NS = 16
_RPT = _B // (_NC * _NS)
_CHUNK = 128
_NCHUNK = _RPT // _CHUNK


def _sc_seg_body(x_hbm, lab_hbm, zeros_hbm, out_hbm,
                 idx0, idx1, idx2, idx3, rows_v, table):
    c = lax.axis_index("c")
    s = lax.axis_index("s")
    wid = s * _NC + c
    base = wid * _RPT

    @pl.when(s == 0)
    def _zero():
        pltpu.sync_copy(zeros_hbm, table)

    pltpu.sync_copy(x_hbm.at[pl.ds(base, _RPT)], rows_v)
    plsc.subcore_barrier()
    idxs = (idx0, idx1, idx2, idx3)
    for k in range(_NCHUNK):
        pltpu.sync_copy(lab_hbm.at[pl.ds(base + k * _CHUNK, _CHUNK)], idxs[k])
        pltpu.sync_copy(rows_v.at[pl.ds(k * _CHUNK, _CHUNK)],
                        table.at[idxs[k]], add=True)
    plsc.subcore_barrier()
    rows_out = _LPAD // _NS
    pltpu.sync_copy(table.at[pl.ds(s * rows_out, rows_out)],
                    out_hbm.at[c, pl.ds(s * rows_out, rows_out)])


def _sc_segment_sum(f_emb, label, zeros):
    mesh = plsc.VectorSubcoreMesh(core_axis_name="c", subcore_axis_name="s")
    run = functools.partial(
        pl.kernel,
        mesh=mesh,
        out_type=jax.ShapeDtypeStruct((_NC, _LPAD, _D), jnp.float32),
        scratch_types=[
            pltpu.VMEM((_CHUNK,), jnp.int32),
            pltpu.VMEM((_CHUNK,), jnp.int32),
            pltpu.VMEM((_CHUNK,), jnp.int32),
            pltpu.VMEM((_CHUNK,), jnp.int32),
            pltpu.VMEM((_RPT, _D), jnp.float32),
            pltpu.VMEM_SHARED((_LPAD, _D), jnp.float32),
        ],
    )(_sc_seg_body)
    return run(f_emb, label, zeros)


def _ce_body(f_ref, lab_ref, labs_ref, psum_ref, tau_ref, out_ref, pn_ref):
    i = pl.program_id(0)

    @pl.when(i == 0)
    def _init():
        rowid = lax.broadcasted_iota(jnp.int32, (_LPAD, _LPAD), 0)
        c = jnp.zeros((_LPAD, 1), jnp.float32)
        for k in range(_B // _LPAD):
            labrow = labs_ref[k].reshape(1, _LPAD)
            c += jnp.sum(jnp.where(rowid == labrow, 1.0, 0.0),
                         axis=1, keepdims=True)
        s = psum_ref[0] + psum_ref[1]
        mean = s / (c + 1e-6)
        mean = jnp.where(c < 0.5, jnp.zeros_like(mean), mean)
        nrm = jnp.sqrt(jnp.sum(mean * mean, axis=1, keepdims=True))
        pn_ref[...] = mean / jnp.maximum(nrm, 1e-6)
        out_ref[...] = jnp.zeros_like(out_ref)

    f = f_ref[...]
    nrm = jnp.sqrt(jnp.sum(f * f, axis=1, keepdims=True))
    fn = f / jnp.maximum(nrm, 1e-6)
    scale = jnp.exp(jnp.clip(tau_ref[...], 0.0, _CLAMP))
    logits = lax.dot_general(
        fn, pn_ref[...], (((1,), (1,)), ((), ())),
        preferred_element_type=jnp.float32,
        precision=lax.Precision.DEFAULT) * scale
    colid = lax.broadcasted_iota(jnp.int32, (_BLK, _LPAD), 1)
    logits = jnp.where(colid < _NUM_LABELS, logits, jnp.float32(-1e30))
    m = jnp.max(logits, axis=1, keepdims=True)
    ez = jnp.sum(jnp.exp(logits - m), axis=1, keepdims=True)
    logz = jnp.log(ez) + m
    lab = lab_ref[...]
    picked = jnp.sum(jnp.where(colid == lab, logits, 0.0), axis=1, keepdims=True)
    out_ref[...] += jnp.sum(logz - picked)

    @pl.when(i == _NSTEPS - 1)
    def _fin():
        out_ref[...] = out_ref[...] * jnp.float32(1.0 / _B)


def _ce_loss(f_emb, label, psum, tau):
    labc = label.reshape(_B, 1)
    labs2 = label.reshape(_B // _LPAD, _LPAD)
    tau2 = tau.reshape(1, 1)
    acc = pl.pallas_call(
        _ce_body,
        grid=(_NSTEPS,),
        in_specs=[
            pl.BlockSpec((_BLK, _D), lambda i: (i, 0)),
            pl.BlockSpec((_BLK, 1), lambda i: (i, 0)),
            pl.BlockSpec((_B // _LPAD, _LPAD), lambda i: (0, 0)),
            pl.BlockSpec((_NC, _LPAD, _D), lambda i: (0, 0, 0)),
            pl.BlockSpec((1, 1), lambda i: (0, 0)),
        ],
        out_specs=pl.BlockSpec((1, 1), lambda i: (0, 0)),
        out_shape=jax.ShapeDtypeStruct((1, 1), jnp.float32),
        scratch_shapes=[pltpu.VMEM((_LPAD, _D), jnp.float32)],
        compiler_params=pltpu.CompilerParams(
            dimension_semantics=("arbitrary",)),
    )(f_emb, labc, labs2, psum, tau2)
    return acc[0, 0]


def kernel(f_emb, label, tau):
    zeros = jnp.zeros((_LPAD, _D), jnp.float32)
    psum = _sc_segment_sum(f_emb, label, zeros)
    return _ce_loss(f_emb, label, psum, tau)


# speedup vs baseline: 1.0933x; 1.0933x over previous
import functools

import jax
import jax.numpy as jnp
from jax import lax
from jax.experimental import pallas as pl
from jax.experimental.pallas import tpu as pltpu
from jax.experimental.pallas import tpu_sc as plsc

_NUM_LABELS = 1000
_CLAMP = 4.6051
_B = 16384
_D = 128
_LPAD = 1024
_BLK = 512
_NSTEPS = _B // _BLK
_NC = 2
_NS = 16
_RPT = _B // (_NC * _NS)
_CHUNK = 128
_NCHUNK = _RPT // _CHUNK


def _sc_seg_body(x_hbm, lab_hbm, zeros_hbm, out_hbm,
                 idx0, idx1, idx2, idx3, rows_v, table):
    c = lax.axis_index("c")
    s = lax.axis_index("s")
    wid = s * _NC + c
    base = wid * _RPT

    @pl.when(s == 0)
    def _zero():
        pltpu.sync_copy(zeros_hbm, table)

    pltpu.sync_copy(x_hbm.at[pl.ds(base, _RPT)], rows_v)
    plsc.subcore_barrier()
    idxs = (idx0, idx1, idx2, idx3)
    for k in range(_NCHUNK):
        pltpu.sync_copy(lab_hbm.at[pl.ds(base + k * _CHUNK, _CHUNK)], idxs[k])
        pltpu.sync_copy(rows_v.at[pl.ds(k * _CHUNK, _CHUNK)],
                        table.at[idxs[k]], add=True)
    plsc.subcore_barrier()
    rows_out = _LPAD // _NS
    pltpu.sync_copy(table.at[pl.ds(s * rows_out, rows_out)],
                    out_hbm.at[c, pl.ds(s * rows_out, rows_out)])


def _sc_segment_sum(f_emb, label, zeros):
    mesh = plsc.VectorSubcoreMesh(core_axis_name="c", subcore_axis_name="s")
    run = functools.partial(
        pl.kernel,
        mesh=mesh,
        out_type=jax.ShapeDtypeStruct((_NC, _LPAD, _D), jnp.float32),
        scratch_types=[
            pltpu.VMEM((_CHUNK,), jnp.int32),
            pltpu.VMEM((_CHUNK,), jnp.int32),
            pltpu.VMEM((_CHUNK,), jnp.int32),
            pltpu.VMEM((_CHUNK,), jnp.int32),
            pltpu.VMEM((_RPT, _D), jnp.float32),
            pltpu.VMEM_SHARED((_LPAD, _D), jnp.float32),
        ],
    )(_sc_seg_body)
    return run(f_emb, label, zeros)


def _ce_body(f_ref, lab_ref, psum_ref, tau_ref, out_ref, pn_ref):
    i = pl.program_id(0)

    @pl.when(i == 0)
    def _init():
        s = psum_ref[0] + psum_ref[1]
        nrm = jnp.sqrt(jnp.sum(s * s, axis=1, keepdims=True))
        pn_ref[...] = s / jnp.maximum(nrm, 1e-6)
        out_ref[...] = jnp.zeros_like(out_ref)

    f = f_ref[...]
    nrm = jnp.sqrt(jnp.sum(f * f, axis=1, keepdims=True))
    fn = f / jnp.maximum(nrm, 1e-6)
    scale = jnp.exp(jnp.clip(tau_ref[...], 0.0, _CLAMP))
    logits = lax.dot_general(
        fn, pn_ref[...], (((1,), (1,)), ((), ())),
        preferred_element_type=jnp.float32,
        precision=lax.Precision.DEFAULT) * scale
    colid = lax.broadcasted_iota(jnp.int32, (_BLK, _LPAD), 1)
    logits = jnp.where(colid < _NUM_LABELS, logits, jnp.float32(-1e30))
    m = jnp.max(logits, axis=1, keepdims=True)
    ez = jnp.sum(jnp.exp(logits - m), axis=1, keepdims=True)
    logz = jnp.log(ez) + m
    lab = lab_ref[...]
    picked = jnp.sum(jnp.where(colid == lab, logits, 0.0), axis=1, keepdims=True)
    out_ref[...] += jnp.sum(logz - picked)

    @pl.when(i == _NSTEPS - 1)
    def _fin():
        out_ref[...] = out_ref[...] * jnp.float32(1.0 / _B)


def _ce_loss(f_emb, label, psum, tau):
    labc = label.reshape(_B, 1)
    tau2 = tau.reshape(1, 1)
    acc = pl.pallas_call(
        _ce_body,
        grid=(_NSTEPS,),
        in_specs=[
            pl.BlockSpec((_BLK, _D), lambda i: (i, 0)),
            pl.BlockSpec((_BLK, 1), lambda i: (i, 0)),
            pl.BlockSpec((_NC, _LPAD, _D), lambda i: (0, 0, 0)),
            pl.BlockSpec((1, 1), lambda i: (0, 0)),
        ],
        out_specs=pl.BlockSpec((1, 1), lambda i: (0, 0)),
        out_shape=jax.ShapeDtypeStruct((1, 1), jnp.float32),
        scratch_shapes=[pltpu.VMEM((_LPAD, _D), jnp.float32)],
        compiler_params=pltpu.CompilerParams(
            dimension_semantics=("arbitrary",)),
    )(f_emb, labc, psum, tau2)
    return acc[0, 0]


def kernel(f_emb, label, tau):
    zeros = jnp.zeros((_LPAD, _D), jnp.float32)
    psum = _sc_segment_sum(f_emb, label, zeros)
    return _ce_loss(f_emb, label, psum, tau)


# speedup vs baseline: 1.1398x; 1.0425x over previous
import functools

import jax
import jax.numpy as jnp
from jax import lax
from jax.experimental import pallas as pl
from jax.experimental.pallas import tpu as pltpu
from jax.experimental.pallas import tpu_sc as plsc

_NUM_LABELS = 1000
_CLAMP = 4.6051
_B = 16384
_D = 128
_LPAD = 1024
_BLK = 512
_NSTEPS = _B // _BLK
_NC = 2
_NS = 16
_RPT = _B // (_NC * _NS)
_CHUNK = 128
_NCHUNK = _RPT // _CHUNK


def _sc_seg_body(x_hbm, lab_hbm, zeros_hbm, out_hbm,
                 idx0, idx1, idx2, idx3, rows_v, table):
    c = lax.axis_index("c")
    s = lax.axis_index("s")
    wid = s * _NC + c
    base = wid * _RPT

    @pl.when(s == 0)
    def _zero():
        pltpu.sync_copy(zeros_hbm, table)

    pltpu.sync_copy(x_hbm.at[pl.ds(base, _RPT)], rows_v)
    plsc.subcore_barrier()
    idxs = (idx0, idx1, idx2, idx3)
    for k in range(_NCHUNK):
        pltpu.sync_copy(lab_hbm.at[pl.ds(base + k * _CHUNK, _CHUNK)], idxs[k])
        pltpu.sync_copy(rows_v.at[pl.ds(k * _CHUNK, _CHUNK)],
                        table.at[idxs[k]], add=True)
    plsc.subcore_barrier()
    rows_out = _LPAD // _NS
    pltpu.sync_copy(table.at[pl.ds(s * rows_out, rows_out)],
                    out_hbm.at[c, pl.ds(s * rows_out, rows_out)])


def _sc_segment_sum(f_emb, label, zeros):
    mesh = plsc.VectorSubcoreMesh(core_axis_name="c", subcore_axis_name="s")
    run = functools.partial(
        pl.kernel,
        mesh=mesh,
        out_type=jax.ShapeDtypeStruct((_NC, _LPAD, _D), jnp.float32),
        scratch_types=[
            pltpu.VMEM((_CHUNK,), jnp.int32),
            pltpu.VMEM((_CHUNK,), jnp.int32),
            pltpu.VMEM((_CHUNK,), jnp.int32),
            pltpu.VMEM((_CHUNK,), jnp.int32),
            pltpu.VMEM((_RPT, _D), jnp.float32),
            pltpu.VMEM_SHARED((_LPAD, _D), jnp.float32),
        ],
    )(_sc_seg_body)
    return run(f_emb, label, zeros)


def _ce_body(f_ref, lab_ref, psum_ref, tau_ref, out_ref, pn_ref):
    i = pl.program_id(0)

    @pl.when(i == 0)
    def _init():
        s = psum_ref[0] + psum_ref[1]
        nrm = jnp.sqrt(jnp.sum(s * s, axis=1, keepdims=True))
        scale = jnp.exp(jnp.clip(tau_ref[...], 0.0, _CLAMP))
        pn_ref[...] = (s * (scale / jnp.maximum(nrm, 1e-6))).astype(jnp.bfloat16)
        out_ref[...] = jnp.zeros_like(out_ref)

    f = f_ref[...]
    nrm = jnp.sqrt(jnp.sum(f * f, axis=1, keepdims=True))
    fn = (f / jnp.maximum(nrm, 1e-6)).astype(jnp.bfloat16)
    padrow = jnp.where(
        lax.broadcasted_iota(jnp.int32, (1, _LPAD), 1) < _NUM_LABELS,
        jnp.float32(0.0), jnp.float32(-1e30))
    logits = lax.dot_general(
        fn, pn_ref[...], (((1,), (1,)), ((), ())),
        preferred_element_type=jnp.float32) + padrow
    colid = lax.broadcasted_iota(jnp.int32, (_BLK, _LPAD), 1)
    m = jnp.max(logits, axis=1, keepdims=True)
    ez = jnp.sum(jnp.exp(logits - m), axis=1, keepdims=True)
    logz = jnp.log(ez) + m
    lab = lab_ref[...]
    picked = jnp.sum(jnp.where(colid == lab, logits, 0.0), axis=1, keepdims=True)
    out_ref[...] += jnp.sum(logz - picked)

    @pl.when(i == _NSTEPS - 1)
    def _fin():
        out_ref[...] = out_ref[...] * jnp.float32(1.0 / _B)


def _ce_loss(f_emb, label, psum, tau):
    labc = label.reshape(_B, 1)
    tau2 = tau.reshape(1, 1)
    acc = pl.pallas_call(
        _ce_body,
        grid=(_NSTEPS,),
        in_specs=[
            pl.BlockSpec((_BLK, _D), lambda i: (i, 0)),
            pl.BlockSpec((_BLK, 1), lambda i: (i, 0)),
            pl.BlockSpec((_NC, _LPAD, _D), lambda i: (0, 0, 0)),
            pl.BlockSpec((1, 1), lambda i: (0, 0)),
        ],
        out_specs=pl.BlockSpec((1, 1), lambda i: (0, 0)),
        out_shape=jax.ShapeDtypeStruct((1, 1), jnp.float32),
        scratch_shapes=[pltpu.VMEM((_LPAD, _D), jnp.bfloat16)],
        compiler_params=pltpu.CompilerParams(
            dimension_semantics=("arbitrary",)),
    )(f_emb, labc, psum, tau2)
    return acc[0, 0]


def kernel(f_emb, label, tau):
    zeros = jnp.zeros((_LPAD, _D), jnp.float32)
    psum = _sc_segment_sum(f_emb, label, zeros)
    return _ce_loss(f_emb, label, psum, tau)
